# Initial kernel scaffold; baseline (speedup 1.0000x reference)
import jax
import jax.numpy as jnp
from jax.experimental import pallas as pl


def kernel(q, k, top_k_instances, top_k_relationships):
    raise NotImplementedError("write your pallas kernel here")


# speedup vs baseline: 3.4594x; 3.4594x over previous
import jax
import jax.numpy as jnp
from jax import lax
from jax.experimental import pallas as pl
from jax.experimental.pallas import tpu as pltpu

N = 4096
D = 768
B = 2
K = 10
R = 5
BR = 256
NRB = N // BR

_NEG = -3e38
_BIG = 1 << 30


def _rowkey_body(q_ref, k_ref, key_ref, m_ref, den_ref):
    qb = q_ref[0]
    kb = k_ref[0]
    s = lax.dot_general(qb, kb, (((1,), (1,)), ((), ())),
                        preferred_element_type=jnp.float32)
    m = jnp.max(s, axis=1, keepdims=True)
    e = jnp.exp(s - m)
    denom = jnp.sum(e, axis=1)
    i = pl.program_id(1)
    row_ids = lax.broadcasted_iota(jnp.int32, (BR, N), 0)
    col_ids = lax.broadcasted_iota(jnp.int32, (BR, N), 1)
    dmask = col_ids == row_ids + i * BR
    dexp = jnp.sum(jnp.where(dmask, e, 0.0), axis=1)
    key_ref[0, 0, 0, :] = dexp / denom
    m_ref[0, 0, 0, :] = m[:, 0]
    den_ref[0, 0, 0, :] = denom


def _rowkey(q, k):
    outs = pl.pallas_call(
        _rowkey_body,
        grid=(B, NRB),
        in_specs=[
            pl.BlockSpec((1, BR, D), lambda b, i: (b, i, 0)),
            pl.BlockSpec((1, N, D), lambda b, i: (b, 0, 0)),
        ],
        out_specs=[pl.BlockSpec((1, 1, 1, BR), lambda b, i: (b, i, 0, 0))] * 3,
        out_shape=[jax.ShapeDtypeStruct((B, NRB, 1, BR), jnp.float32)] * 3,
    )(q, k)
    return tuple(o.reshape(B, N) for o in outs)


def _select_body(key_ref, m_ref, den_ref, q_ref, k_ref, tk_ref, obj_refs,
                 rel_refs):
    keym = key_ref[0]
    qb = q_ref[0]
    kb = k_ref[0]
    LANES = N // 8
    sub_i = lax.broadcasted_iota(jnp.int32, (8, LANES), 0)
    lane_i = lax.broadcasted_iota(jnp.int32, (8, LANES), 1)
    flat_i = sub_i * LANES + lane_i

    kcur = keym
    selmask = jnp.zeros((8, LANES), dtype=jnp.bool_)
    for _ in range(K):
        mx = jnp.max(kcur)
        idx = jnp.min(jnp.where(kcur == mx, flat_i, _BIG))
        hit = flat_i == idx
        selmask = selmask | hit
        kcur = jnp.where(hit, -2.0, kcur)

    tks = []
    msk = selmask
    for _ in range(K):
        mn = jnp.min(jnp.where(msk, flat_i, _BIG))
        tks.append(mn)
        msk = msk & (flat_i != mn)

    lane16 = lax.broadcasted_iota(jnp.int32, (1, 16), 1)
    tvec = jnp.full((1, 16), N, dtype=jnp.int32)
    for j in range(K):
        tvec = jnp.where(lane16 == j, tks[j], tvec)
    tk_ref[0, 0, :] = tvec[0]

    row16 = lax.broadcasted_iota(jnp.int32, (16, N), 0)
    colN = lax.broadcasted_iota(jnp.int32, (16, N), 1)
    tfull = jnp.full((16, N), N, dtype=jnp.int32)
    for j in range(K):
        tfull = jnp.where(row16 == j, tks[j], tfull)
    ohb_rows = colN == tfull
    oh = ohb_rows.astype(jnp.float32)
    q_top = jnp.dot(oh, qb, preferred_element_type=jnp.float32)
    k_top = jnp.dot(oh, kb, preferred_element_type=jnp.float32)
    m_top = jnp.sum(jnp.where(ohb_rows, jnp.broadcast_to(m_ref[0], (16, N)),
                              0.0), axis=1, keepdims=True)
    den_top = jnp.sum(jnp.where(ohb_rows, jnp.broadcast_to(den_ref[0], (16, N)),
                                0.0), axis=1, keepdims=True)

    s10 = lax.dot_general(q_top, k_top, (((1,), (1,)), ((), ())),
                          preferred_element_type=jnp.float32)
    rs10 = jnp.exp(s10 - m_top) / den_top
    r_i = lax.broadcasted_iota(jnp.int32, (16, 16), 0)
    c_i = lax.broadcasted_iota(jnp.int32, (16, 16), 1)
    valid = (r_i < K) & (c_i < K)
    rs10 = jnp.where(valid, rs10, _NEG)

    scur = rs10
    sel = jnp.zeros((16, 16), dtype=jnp.bool_)
    for _ in range(R):
        mx = jnp.max(scur, axis=1, keepdims=True)
        cj = jnp.min(jnp.where(scur == mx, c_i, _BIG), axis=1, keepdims=True)
        hit = c_i == cj
        sel = sel | hit
        scur = jnp.where(hit, _NEG, scur)

    selF = sel.astype(jnp.float32)
    lt = (r_i <= c_i).astype(jnp.float32)
    prefix = jnp.dot(selF, lt, preferred_element_type=jnp.float32)

    trow = jnp.broadcast_to(tvec, (16, 16))
    for j in range(R):
        ohb = sel & (prefix == (j + 1.0))
        ohf = ohb.astype(jnp.float32)
        objid = jnp.sum(jnp.where(ohb, trow, 0), axis=1)
        obj_refs[j][0, 0, :] = objid
        eobj = jnp.dot(ohf, q_top, preferred_element_type=jnp.float32)
        rel0 = q_top + eobj
        mean = jnp.mean(rel0, axis=1, keepdims=True)
        var = jnp.mean((rel0 - mean) ** 2, axis=1, keepdims=True)
        rel_refs[j][0] = (rel0 - mean) / jnp.sqrt(var + 1e-5)


def _select_wrap(key_ref, m_ref, den_ref, q_ref, k_ref, tk_ref,
                 o0, o1, o2, o3, o4, e0, e1, e2, e3, e4):
    _select_body(key_ref, m_ref, den_ref, q_ref, k_ref, tk_ref,
                 [o0, o1, o2, o3, o4], [e0, e1, e2, e3, e4])


def _select(key, m, den, q, k):
    key3 = key.reshape(B, 8, N // 8)
    m3 = m.reshape(B, 1, N)
    den3 = den.reshape(B, 1, N)
    outs = pl.pallas_call(
        _select_wrap,
        grid=(B,),
        in_specs=[
            pl.BlockSpec((1, 8, N // 8), lambda b: (b, 0, 0)),
            pl.BlockSpec((1, 1, N), lambda b: (b, 0, 0)),
            pl.BlockSpec((1, 1, N), lambda b: (b, 0, 0)),
            pl.BlockSpec((1, N, D), lambda b: (b, 0, 0)),
            pl.BlockSpec((1, N, D), lambda b: (b, 0, 0)),
        ],
        out_specs=[pl.BlockSpec((1, 1, 16), lambda b: (b, 0, 0))]
        + [pl.BlockSpec((1, 1, 16), lambda b: (b, 0, 0))] * R
        + [pl.BlockSpec((1, 16, D), lambda b: (b, 0, 0))] * R,
        out_shape=[jax.ShapeDtypeStruct((B, 1, 16), jnp.int32)]
        + [jax.ShapeDtypeStruct((B, 1, 16), jnp.int32)] * R
        + [jax.ShapeDtypeStruct((B, 16, D), jnp.float32)] * R,
    )(key3, m3, den3, q, k)
    tk = outs[0][:, 0, :]
    objs = [o[:, 0, :] for o in outs[1:1 + R]]
    rels = outs[1 + R:]
    return tk, objs, rels


def kernel(q, k, top_k_instances, top_k_relationships):
    del top_k_instances, top_k_relationships
    key, m, den = _rowkey(q, k)
    tk, objs, rels = _select(key, m, den, q, k)
    obj50 = jnp.stack(objs, axis=-1)[:, :K, :].reshape(B, K * R)
    sub50 = jnp.repeat(tk[:, :K], R, axis=1)
    bids = jnp.broadcast_to(jnp.arange(B, dtype=jnp.int32)[:, None], (B, K * R))
    soi = jnp.stack([bids, sub50, obj50], axis=-1)
    rel = jnp.stack(rels, axis=2)[:, :K].reshape(B, K * R, D)
    return soi, rel
